# Initial kernel scaffold; baseline (speedup 1.0000x reference)
import jax
import jax.numpy as jnp
from jax.experimental import pallas as pl


def kernel(is_target_mask, embedding_weight):
    raise NotImplementedError("write your pallas kernel here")


# speedup vs baseline: 6.8325x; 6.8325x over previous
import jax
import jax.numpy as jnp
from jax.experimental import pallas as pl

B, L, D = 4096, 200, 128
N = B * L
RB = 25600


def _body(mask_ref, w_ref, out_ref):
    m = mask_ref[...]
    w0 = w_ref[0:1, :]
    w1 = w_ref[1:2, :]
    out_ref[...] = jnp.where(m != 0, w1, w0)


def kernel(is_target_mask, embedding_weight):
    mask2d = is_target_mask.astype(jnp.int32).reshape(N, 1)
    grid = (N // RB,)
    out = pl.pallas_call(
        _body,
        grid=grid,
        in_specs=[
            pl.BlockSpec((RB, 1), lambda i: (i, 0)),
            pl.BlockSpec((2, D), lambda i: (0, 0)),
        ],
        out_specs=pl.BlockSpec((RB, D), lambda i: (i, 0)),
        out_shape=jax.ShapeDtypeStruct((N, D), jnp.float32),
    )(mask2d, embedding_weight)
    return out.reshape(B, L, D)


# speedup vs baseline: 6.8348x; 1.0003x over previous
import jax
import jax.numpy as jnp
from jax.experimental import pallas as pl
from jax.experimental.pallas import tpu as pltpu

B, L, D = 4096, 200, 128
N = B * L
RB = 25600


def _body(mask_ref, w_ref, out_ref):
    m = mask_ref[...]
    w0 = w_ref[0:1, :]
    w1 = w_ref[1:2, :]
    out_ref[...] = jnp.where(m != 0, w1, w0)


def kernel(is_target_mask, embedding_weight):
    mask2d = is_target_mask.astype(jnp.int32).reshape(N, 1)
    grid = (N // RB,)
    out = pl.pallas_call(
        _body,
        grid=grid,
        in_specs=[
            pl.BlockSpec((RB, 1), lambda i: (i, 0)),
            pl.BlockSpec((2, D), lambda i: (0, 0)),
        ],
        out_specs=pl.BlockSpec((RB, D), lambda i: (i, 0)),
        out_shape=jax.ShapeDtypeStruct((N, D), jnp.float32),
        compiler_params=pltpu.CompilerParams(
            dimension_semantics=("parallel",),
        ),
    )(mask2d, embedding_weight)
    return out.reshape(B, L, D)
